# initial kernel scaffold (unmeasured)
import jax
import jax.numpy as jnp
from jax import lax
from jax.experimental import pallas as pl
from jax.experimental.pallas import tpu as pltpu

N_DEV = 4


def kernel(x, W):
    t, d = x.shape
    _, v_loc = W.shape

    def body(x_ref, w_ref, out_ref, comm_ref, send_sems, recv_sems):
        my_pos = lax.axis_index("i")
        left = (my_pos - 1) % N_DEV
        right = (my_pos + 1) % N_DEV

        barrier_sem = pltpu.get_barrier_semaphore()
        for nbr in [left, right]:
            pl.semaphore_signal(
                barrier_sem, inc=1,
                device_id=(nbr,), device_id_type=pl.DeviceIdType.MESH,
            )
        pl.semaphore_wait(barrier_sem, 2)

        logits = jnp.dot(x_ref[:, :], w_ref[:, :],
                         preferred_element_type=jnp.float32)
        out_ref[:, pl.ds(my_pos * v_loc, v_loc)] = logits
        comm_ref[0, :, :] = logits

        for h in range(N_DEV - 1):
            send_slot = h % 2
            recv_slot = (h + 1) % 2
            rdma = pltpu.make_async_remote_copy(
                src_ref=comm_ref.at[send_slot],
                dst_ref=comm_ref.at[recv_slot],
                send_sem=send_sems.at[send_slot],
                recv_sem=recv_sems.at[recv_slot],
                device_id=(right,),
                device_id_type=pl.DeviceIdType.MESH,
            )
            rdma.start()
            rdma.wait()

            origin = (my_pos - h - 1) % N_DEV
            out_ref[:, pl.ds(origin * v_loc, v_loc)] = comm_ref[recv_slot, :, :]

        full = out_ref[:, :]
        m = jnp.max(full, axis=-1, keepdims=True)
        e = jnp.exp(full - m)
        out_ref[:, :] = e / jnp.sum(e, axis=-1, keepdims=True)

    return pl.pallas_call(
        body,
        out_shape=jax.ShapeDtypeStruct((t, N_DEV * v_loc), jnp.float32),
        in_specs=[
            pl.BlockSpec(memory_space=pltpu.VMEM),
            pl.BlockSpec(memory_space=pltpu.VMEM),
        ],
        out_specs=pl.BlockSpec(memory_space=pltpu.VMEM),
        scratch_shapes=[
            pltpu.VMEM((2, t, v_loc), jnp.float32),
            pltpu.SemaphoreType.DMA((2,)),
            pltpu.SemaphoreType.DMA((2,)),
        ],
        compiler_params=pltpu.CompilerParams(collective_id=0),
    )(x, W)


# baseline (device time: 171328 ns/iter reference)
import jax
import jax.numpy as jnp
from jax import lax
from jax.experimental import pallas as pl
from jax.experimental.pallas import tpu as pltpu

N_DEV = 4


def kernel(x, W):
    t, d = x.shape
    _, v_loc = W.shape

    def body(x_ref, w_ref, out_ref, comm_ref, send_sems, recv_sems):
        my_pos = lax.axis_index("i")
        left = (my_pos - 1) % N_DEV
        right = (my_pos + 1) % N_DEV

        barrier_sem = pltpu.get_barrier_semaphore()
        for nbr in [left, right]:
            pl.semaphore_signal(
                barrier_sem, inc=1,
                device_id=(nbr,), device_id_type=pl.DeviceIdType.MESH,
            )
        pl.semaphore_wait(barrier_sem, 2)

        logits = jnp.dot(x_ref[:, :], w_ref[:, :],
                         preferred_element_type=jnp.float32)
        out_ref[:, pl.ds(my_pos * v_loc, v_loc)] = logits
        comm_ref[0, :, :] = logits

        for h in range(N_DEV - 1):
            send_slot = h % 2
            recv_slot = (h + 1) % 2
            rdma = pltpu.make_async_remote_copy(
                src_ref=comm_ref.at[send_slot],
                dst_ref=comm_ref.at[recv_slot],
                send_sem=send_sems.at[send_slot],
                recv_sem=recv_sems.at[recv_slot],
                device_id=(right,),
                device_id_type=pl.DeviceIdType.MESH,
            )
            rdma.start()
            rdma.wait()

            origin = (my_pos - h - 1) % N_DEV
            out_ref[:, pl.ds(origin * v_loc, v_loc)] = comm_ref[recv_slot, :, :]

        full = out_ref[:, :]
        m = jnp.max(full, axis=-1, keepdims=True)
        e = jnp.exp(full - m)
        out_ref[:, :] = e / jnp.sum(e, axis=-1, keepdims=True)

    return pl.pallas_call(
        body,
        out_shape=jax.ShapeDtypeStruct((t, N_DEV * v_loc), jnp.float32),
        in_specs=[
            pl.BlockSpec(memory_space=pltpu.VMEM),
            pl.BlockSpec(memory_space=pltpu.VMEM),
        ],
        out_specs=pl.BlockSpec(memory_space=pltpu.VMEM),
        scratch_shapes=[
            pltpu.VMEM((2, t, v_loc), jnp.float32),
            pltpu.SemaphoreType.DMA((2,)),
            pltpu.SemaphoreType.DMA((2,)),
        ],
        compiler_params=pltpu.CompilerParams(
            collective_id=0,
            vmem_limit_bytes=100 * 1024 * 1024,
        ),
    )(x, W)


# device time: 103724 ns/iter; 1.6518x vs baseline; 1.6518x over previous
import jax
import jax.numpy as jnp
from jax import lax
from jax.experimental import pallas as pl
from jax.experimental.pallas import tpu as pltpu

N_DEV = 4


def kernel(x, W):
    t, d = x.shape
    _, v_loc = W.shape
    v_half = v_loc // 2

    def body(x_ref, w_ref, out_ref, comm_r, comm_l,
             send_r, recv_r, send_l, recv_l):
        my_pos = lax.axis_index("i")
        left = (my_pos - 1) % N_DEV
        right = (my_pos + 1) % N_DEV

        barrier_sem = pltpu.get_barrier_semaphore()
        for nbr in [left, right]:
            pl.semaphore_signal(
                barrier_sem, inc=1,
                device_id=(nbr,), device_id_type=pl.DeviceIdType.MESH,
            )
        pl.semaphore_wait(barrier_sem, 2)

        logits = jnp.dot(x_ref[:, :], w_ref[:, :],
                         preferred_element_type=jnp.float32)
        out_ref[:, pl.ds(my_pos * v_loc, v_loc)] = logits
        comm_r[0, :, :] = logits[:, :v_half]
        comm_l[0, :, :] = logits[:, v_half:]

        for h in range(N_DEV - 1):
            s = h % 2
            r = (h + 1) % 2
            rdma_r = pltpu.make_async_remote_copy(
                src_ref=comm_r.at[s], dst_ref=comm_r.at[r],
                send_sem=send_r.at[s], recv_sem=recv_r.at[r],
                device_id=(right,), device_id_type=pl.DeviceIdType.MESH,
            )
            rdma_l = pltpu.make_async_remote_copy(
                src_ref=comm_l.at[s], dst_ref=comm_l.at[r],
                send_sem=send_l.at[s], recv_sem=recv_l.at[r],
                device_id=(left,), device_id_type=pl.DeviceIdType.MESH,
            )
            rdma_r.start()
            rdma_l.start()
            rdma_r.wait()
            rdma_l.wait()

            origin_r = (my_pos - h - 1) % N_DEV
            origin_l = (my_pos + h + 1) % N_DEV
            out_ref[:, pl.ds(origin_r * v_loc, v_half)] = comm_r[r, :, :]
            out_ref[:, pl.ds(origin_l * v_loc + v_half, v_half)] = \
                comm_l[r, :, :]

        full = out_ref[:, :]
        m = jnp.max(full, axis=-1, keepdims=True)
        e = jnp.exp(full - m)
        out_ref[:, :] = e / jnp.sum(e, axis=-1, keepdims=True)

    return pl.pallas_call(
        body,
        out_shape=jax.ShapeDtypeStruct((t, N_DEV * v_loc), jnp.float32),
        in_specs=[
            pl.BlockSpec(memory_space=pltpu.VMEM),
            pl.BlockSpec(memory_space=pltpu.VMEM),
        ],
        out_specs=pl.BlockSpec(memory_space=pltpu.VMEM),
        scratch_shapes=[
            pltpu.VMEM((2, t, v_half), jnp.float32),
            pltpu.VMEM((2, t, v_half), jnp.float32),
            pltpu.SemaphoreType.DMA((2,)),
            pltpu.SemaphoreType.DMA((2,)),
            pltpu.SemaphoreType.DMA((2,)),
            pltpu.SemaphoreType.DMA((2,)),
        ],
        compiler_params=pltpu.CompilerParams(
            collective_id=0,
            vmem_limit_bytes=100 * 1024 * 1024,
        ),
    )(x, W)


# device time: 101098 ns/iter; 1.6947x vs baseline; 1.0260x over previous
import jax
import jax.numpy as jnp
from jax import lax
from jax.experimental import pallas as pl
from jax.experimental.pallas import tpu as pltpu

N_DEV = 4


def kernel(x, W):
    t, d = x.shape
    _, v_loc = W.shape
    v_half = v_loc // 2

    def body(x_ref, w_ref, out_ref, comm_r, comm_l,
             send_r, recv_r, send_l, recv_l):
        my_pos = lax.axis_index("i")
        left = (my_pos - 1) % N_DEV
        right = (my_pos + 1) % N_DEV

        barrier_sem = pltpu.get_barrier_semaphore()
        for nbr in [left, right]:
            pl.semaphore_signal(
                barrier_sem, inc=1,
                device_id=(nbr,), device_id_type=pl.DeviceIdType.MESH,
            )
        pl.semaphore_wait(barrier_sem, 2)

        def mk(comm, sends, recvs, h, dev):
            return pltpu.make_async_remote_copy(
                src_ref=comm.at[h], dst_ref=comm.at[h + 1],
                send_sem=sends.at[h], recv_sem=recvs.at[h],
                device_id=(dev,), device_id_type=pl.DeviceIdType.MESH,
            )

        logits = jnp.dot(x_ref[:, :], w_ref[:, :],
                         preferred_element_type=jnp.float32)
        comm_r[0, :, :] = logits[:, :v_half]
        comm_l[0, :, :] = logits[:, v_half:]
        ring_r = [mk(comm_r, send_r, recv_r, 0, right)]
        ring_l = [mk(comm_l, send_l, recv_l, 0, left)]
        ring_r[0].start()
        ring_l[0].start()

        m0 = jnp.max(logits, axis=-1, keepdims=True)
        e0 = jnp.exp(logits - m0)
        stats = [(m0, jnp.sum(e0, axis=-1, keepdims=True))]
        regions = [(my_pos * v_loc, v_loc)]
        out_ref[:, pl.ds(my_pos * v_loc, v_loc)] = e0

        for h in range(N_DEV - 1):
            ring_r[h].wait_recv()
            if h < N_DEV - 2:
                ring_r.append(mk(comm_r, send_r, recv_r, h + 1, right))
                ring_r[h + 1].start()
            ring_l[h].wait_recv()
            if h < N_DEV - 2:
                ring_l.append(mk(comm_l, send_l, recv_l, h + 1, left))
                ring_l[h + 1].start()
            ring_r[h].wait_send()
            ring_l[h].wait_send()

            origin_r = (my_pos - h - 1) % N_DEV
            origin_l = (my_pos + h + 1) % N_DEV
            piece_r = comm_r[h + 1, :, :]
            piece_l = comm_l[h + 1, :, :]
            mr = jnp.max(piece_r, axis=-1, keepdims=True)
            er = jnp.exp(piece_r - mr)
            stats.append((mr, jnp.sum(er, axis=-1, keepdims=True)))
            regions.append((origin_r * v_loc, v_half))
            out_ref[:, pl.ds(origin_r * v_loc, v_half)] = er
            ml = jnp.max(piece_l, axis=-1, keepdims=True)
            el = jnp.exp(piece_l - ml)
            stats.append((ml, jnp.sum(el, axis=-1, keepdims=True)))
            regions.append((origin_l * v_loc + v_half, v_half))
            out_ref[:, pl.ds(origin_l * v_loc + v_half, v_half)] = el

        m = stats[0][0]
        for mp, _ in stats[1:]:
            m = jnp.maximum(m, mp)
        z = stats[0][1] * jnp.exp(stats[0][0] - m)
        for mp, sp in stats[1:]:
            z = z + sp * jnp.exp(mp - m)
        inv_z = 1.0 / z
        for (mp, _), (start, width) in zip(stats, regions):
            scale = jnp.exp(mp - m) * inv_z
            out_ref[:, pl.ds(start, width)] = (
                out_ref[:, pl.ds(start, width)] * scale
            )

    return pl.pallas_call(
        body,
        out_shape=jax.ShapeDtypeStruct((t, N_DEV * v_loc), jnp.float32),
        in_specs=[
            pl.BlockSpec(memory_space=pltpu.VMEM),
            pl.BlockSpec(memory_space=pltpu.VMEM),
        ],
        out_specs=pl.BlockSpec(memory_space=pltpu.VMEM),
        scratch_shapes=[
            pltpu.VMEM((N_DEV, t, v_half), jnp.float32),
            pltpu.VMEM((N_DEV, t, v_half), jnp.float32),
            pltpu.SemaphoreType.DMA((N_DEV - 1,)),
            pltpu.SemaphoreType.DMA((N_DEV - 1,)),
            pltpu.SemaphoreType.DMA((N_DEV - 1,)),
            pltpu.SemaphoreType.DMA((N_DEV - 1,)),
        ],
        compiler_params=pltpu.CompilerParams(
            collective_id=0,
            vmem_limit_bytes=100 * 1024 * 1024,
        ),
    )(x, W)
